# baseline (speedup 1.0000x reference)
import jax
import jax.numpy as jnp
import numpy as np
from jax.experimental import pallas as pl
from jax.experimental.pallas import tpu as pltpu
from jax.sharding import Mesh, PartitionSpec as P

_BLK = 4096


def _prep_kernel(prog_ref, emb_ref, libw_ref, libb_ref, w_ref, c_ref):
    p = jax.nn.softmax(prog_ref[...], axis=-1)
    w = jnp.dot(p, libw_ref[...], preferred_element_type=jnp.float32)
    b = jnp.dot(p, libb_ref[...], preferred_element_type=jnp.float32)
    w_ref[...] = w
    c_ref[...] = emb_ref[...] * w + b


def _exec_kernel(w_ref, c_ref, state_ref, out_ref):
    w = w_ref[...]
    c = c_ref[...]
    x = state_ref[...]
    for t in range(w.shape[0]):
        x = jnp.tanh(x * w[t][None, :] + c[t][None, :])
    out_ref[...] = x


def _run(state, program, step_emb, lib_W, lib_b):
    batch, d = state.shape
    s, prims = program.shape
    blk = min(_BLK, batch)
    w, c = pl.pallas_call(
        _prep_kernel,
        out_shape=(
            jax.ShapeDtypeStruct((s, d), jnp.float32),
            jax.ShapeDtypeStruct((s, d), jnp.float32),
        ),
    )(program, step_emb, lib_W, lib_b)
    return pl.pallas_call(
        _exec_kernel,
        grid=(batch // blk,),
        in_specs=[
            pl.BlockSpec((s, d), lambda i: (0, 0)),
            pl.BlockSpec((s, d), lambda i: (0, 0)),
            pl.BlockSpec((blk, d), lambda i: (i, 0)),
        ],
        out_specs=pl.BlockSpec((blk, d), lambda i: (i, 0)),
        out_shape=jax.ShapeDtypeStruct((batch, d), jnp.float32),
        compiler_params=pltpu.CompilerParams(
            dimension_semantics=("parallel",),
        ),
    )(w, c, state)


def kernel(state, program, step_emb, lib_W, lib_b):
    batch = state.shape[0]
    devs = jax.devices()
    nd = len(devs)
    if nd > 1 and batch % (nd * 8) == 0:
        mesh = Mesh(np.array(devs), ("b",))
        rep = P(None, None)
        fn = jax.shard_map(
            _run,
            mesh=mesh,
            in_specs=(P("b", None), rep, rep, rep, rep),
            out_specs=P("b", None),
            check_vma=False,
        )
        final = fn(state, program, step_emb, lib_W, lib_b)
    else:
        final = _run(state, program, step_emb, lib_W, lib_b)
    return (final, program)


# speedup vs baseline: 7.8209x; 7.8209x over previous
import jax
import jax.numpy as jnp
from jax.experimental import pallas as pl
from jax.experimental.pallas import tpu as pltpu

_BLK = 4096


def _exec_kernel(prog_ref, emb_ref, libw_ref, libb_ref, state_ref, out_ref,
                 w_ref, c_ref):
    @pl.when(pl.program_id(0) == 0)
    def _prep():
        p = jax.nn.softmax(prog_ref[...], axis=-1)
        w = jnp.dot(p, libw_ref[...], preferred_element_type=jnp.float32)
        b = jnp.dot(p, libb_ref[...], preferred_element_type=jnp.float32)
        w_ref[...] = w
        c_ref[...] = emb_ref[...] * w + b

    w = w_ref[...]
    c = c_ref[...]
    x = state_ref[...]
    for t in range(w.shape[0]):
        x = jnp.tanh(x * w[t][None, :] + c[t][None, :])
    out_ref[...] = x


def kernel(state, program, step_emb, lib_W, lib_b):
    batch, d = state.shape
    s, prims = program.shape
    blk = min(_BLK, batch)
    rep2 = lambda i: (0, 0)
    final = pl.pallas_call(
        _exec_kernel,
        grid=(batch // blk,),
        in_specs=[
            pl.BlockSpec((s, prims), rep2),
            pl.BlockSpec((s, d), rep2),
            pl.BlockSpec((prims, d), rep2),
            pl.BlockSpec((prims, d), rep2),
            pl.BlockSpec((blk, d), lambda i: (i, 0)),
        ],
        out_specs=pl.BlockSpec((blk, d), lambda i: (i, 0)),
        out_shape=jax.ShapeDtypeStruct((batch, d), jnp.float32),
        scratch_shapes=[
            pltpu.VMEM((s, d), jnp.float32),
            pltpu.VMEM((s, d), jnp.float32),
        ],
    )(program, step_emb, lib_W, lib_b, state)
    return (final, program)
